# Initial kernel scaffold; baseline (speedup 1.0000x reference)
import jax
import jax.numpy as jnp
from jax.experimental import pallas as pl


def kernel(x, x_shf, edge_index, sub_node_list, positive_ind, negative_ind, batch_labels, enc_W1, enc_b1, enc_W2, enc_b2, dec_W1, dec_b1, dec_W2, dec_b2, be_W1, be_b1, be_W2, be_b2, bd_W1, bd_b1, bd_W2, bd_b2, bil_W, bil_b):
    raise NotImplementedError("write your pallas kernel here")


# speedup vs baseline: 6.9951x; 6.9951x over previous
import functools

import jax
import jax.numpy as jnp
from jax import lax
from jax.experimental import pallas as pl
from jax.experimental.pallas import tpu as pltpu
from jax.experimental.pallas import tpu_sc as plsc

_N = 10000
_E = 320000
_D = 128
_K = 32
_P = 2048
_CHUNK = 125
_NROWS2D = _E // _CHUNK
_NC, _NS = 2, 16
_NW = _NC * _NS
_PER_TILE = _NROWS2D // _NW
_ZROWS = 400
_NZCHUNK = _N // _ZROWS
_PPT = _P // _NW
_BN = 2000


def _make_segsum(width):
    wpv = width // 16
    mesh = plsc.VectorSubcoreMesh(core_axis_name="c", subcore_axis_name="s", num_cores=_NC, num_subcores=_NS)

    @functools.partial(
        pl.kernel,
        out_type=(
            jax.ShapeDtypeStruct((_N, width), jnp.float32),
            jax.ShapeDtypeStruct((_N, width), jnp.float32),
        ),
        mesh=mesh,
        compiler_params=pltpu.CompilerParams(use_tc_tiling_on_sc=False),
        scratch_types=[
            pltpu.VMEM((_PER_TILE, _CHUNK), jnp.int32),
            pltpu.VMEM((_PER_TILE, _CHUNK), jnp.int32),
            pltpu.VMEM((_CHUNK, width), jnp.float32),
            pltpu.VMEM((_ZROWS, width), jnp.float32),
            pltpu.VMEM_SHARED((_N, width), jnp.float32),
            pltpu.SemaphoreType.DMA,
        ],
    )
    def seg(y_hbm, src_hbm, dst_hbm, out0_hbm, out1_hbm,
            src_v, dst_v, rows_v, zbuf, acc_sh, sem):
        cid = lax.axis_index("c")
        sid = lax.axis_index("s")
        wid = cid * _NS + sid

        def zstore(i, carry):
            r = i // wpv
            c = i % wpv
            zbuf[r, pl.ds(c * 16, 16)] = jnp.zeros((16,), jnp.float32)
            return carry

        lax.fori_loop(0, _ZROWS * wpv, zstore, 0)
        for z in range((_NZCHUNK + _NS - 1) // _NS):
            cb = (z * _NS + sid) * _ZROWS

            @pl.when(z * _NS + sid < _NZCHUNK)
            def _():
                pltpu.sync_copy(zbuf, acc_sh.at[pl.ds(cb, _ZROWS)])

        pltpu.sync_copy(src_hbm.at[pl.ds(wid * _PER_TILE, _PER_TILE)], src_v)
        pltpu.sync_copy(dst_hbm.at[pl.ds(wid * _PER_TILE, _PER_TILE)], dst_v)
        plsc.subcore_barrier()

        def body(j, carry):
            pltpu.async_copy(y_hbm.at[src_v.at[j]], rows_v, sem).wait()
            pltpu.sync_copy(rows_v, acc_sh.at[dst_v.at[j]], add=True)
            return carry

        lax.fori_loop(0, _PER_TILE, body, 0)
        plsc.subcore_barrier()

        for z in range((_NZCHUNK + _NS - 1) // _NS):
            cb = (z * _NS + sid) * _ZROWS
            rows = pl.ds(cb, _ZROWS)

            @pl.when((z * _NS + sid < _NZCHUNK) & (cid == 0))
            def _():
                pltpu.sync_copy(acc_sh.at[rows], out0_hbm.at[rows])

            @pl.when((z * _NS + sid < _NZCHUNK) & (cid == 1))
            def _():
                pltpu.sync_copy(acc_sh.at[rows], out1_hbm.at[rows])

    return seg


def _make_pair_gather():
    mesh = plsc.VectorSubcoreMesh(core_axis_name="c", subcore_axis_name="s", num_cores=_NC, num_subcores=_NS)

    @functools.partial(
        pl.kernel,
        out_type=(
            jax.ShapeDtypeStruct((_P, 64), jnp.float32),
            jax.ShapeDtypeStruct((_P, 64), jnp.float32),
        ),
        mesh=mesh,
        compiler_params=pltpu.CompilerParams(use_tc_tiling_on_sc=False),
        scratch_types=[
            pltpu.VMEM((_PPT,), jnp.int32),
            pltpu.VMEM((_PPT, 64), jnp.float32),
            pltpu.SemaphoreType.DMA,
        ],
    )
    def g(zsub_hbm, pos_hbm, neg_hbm, opos_hbm, oneg_hbm, idx_v, rows_v, sem):
        cid = lax.axis_index("c")
        sid = lax.axis_index("s")
        wid = cid * _NS + sid
        sl = pl.ds(wid * _PPT, _PPT)
        pltpu.sync_copy(pos_hbm.at[sl], idx_v)
        pltpu.async_copy(zsub_hbm.at[idx_v], rows_v, sem).wait()
        pltpu.sync_copy(rows_v, opos_hbm.at[sl])
        pltpu.sync_copy(neg_hbm.at[sl], idx_v)
        pltpu.async_copy(zsub_hbm.at[idx_v], rows_v, sem).wait()
        pltpu.sync_copy(rows_v, oneg_hbm.at[sl])

    return g


_seg64 = _make_segsum(64)
_seg48 = _make_segsum(48)
_pair_gather = _make_pair_gather()


def _row_spec(c):
    return pl.BlockSpec((_BN, c), lambda i: (i, 0))


def _full_spec(shape):
    return pl.BlockSpec(shape, lambda i: (0,) * len(shape))


def _tc_call(body, out_shapes, in_specs, out_specs):
    return pl.pallas_call(
        body,
        grid=(_N // _BN,),
        out_shape=out_shapes,
        in_specs=in_specs,
        out_specs=out_specs,
    )


def _pre_body(x_ref, xs_ref, w_ref, ox_ref, os_ref):
    ox_ref[...] = jnp.dot(x_ref[...], w_ref[...], preferred_element_type=jnp.float32)
    os_ref[...] = jnp.dot(xs_ref[...], w_ref[...], preferred_element_type=jnp.float32)


def _mid1_body(yx_ref, ys_ref, p0x_ref, p1x_ref, p0s_ref, p1s_ref, b1_ref,
               w2_ref, o_ref):
    b = b1_ref[...]
    hx = jnp.maximum(yx_ref[...] + p0x_ref[...] + p1x_ref[...] + b, 0.0)
    hs = jnp.maximum(ys_ref[...] + p0s_ref[...] + p1s_ref[...] + b, 0.0)
    a = jnp.dot(hx, w2_ref[...], preferred_element_type=jnp.float32)
    c = jnp.dot(hs, w2_ref[...], preferred_element_type=jnp.float32)
    o_ref[...] = jnp.concatenate([a, c], axis=1)


def _mid2_body(y2_ref, q0_ref, q1_ref, b2_ref, bl_ref, bw1_ref, bb1_ref,
               bw2_ref, bb2_ref, zb_ref, znb_ref):
    b = b2_ref[...]
    z = y2_ref[...] + q0_ref[...] + q1_ref[...] + jnp.concatenate([b, b], axis=1)
    zb_ref[...] = z
    e = jnp.dot(bl_ref[...], bw1_ref[...], preferred_element_type=jnp.float32)
    e = jnp.maximum(e + bb1_ref[...], 0.0)
    be = jnp.dot(e, bw2_ref[...], preferred_element_type=jnp.float32) + bb2_ref[...]
    znb_ref[...] = jnp.concatenate([z[:, :32], be], axis=1)


def _poolfin_body(a0_ref, a1_ref, bw1_ref, bb1_ref, bw2_ref, bb2_ref,
                  zs_ref, lb_ref):
    zs = (a0_ref[...] + a1_ref[...]) * (1.0 / _K)
    zs_ref[...] = zs
    t = jnp.dot(zs[:, :32], bw1_ref[...], preferred_element_type=jnp.float32)
    t = jnp.maximum(t + bb1_ref[...], 0.0)
    lb_ref[...] = jnp.dot(t, bw2_ref[...], preferred_element_type=jnp.float32) + bb2_ref[...]


def _mid3_body(znb_ref, d0_ref, d1_ref, w_ref, b_ref, o_ref):
    agg = znb_ref[...] + d0_ref[...] + d1_ref[...]
    h = jnp.dot(agg, w_ref[...], preferred_element_type=jnp.float32) + b_ref[...]
    o_ref[...] = jnp.maximum(h, 0.0)


def _fin_body(h_ref, e0_ref, e1_ref, w_ref, b_ref, o_ref):
    agg = h_ref[...] + e0_ref[...] + e1_ref[...]
    o_ref[...] = jnp.dot(agg, w_ref[...], preferred_element_type=jnp.float32) + b_ref[...]


def _bil_body(gp_ref, gn_ref, w_ref, bb_ref, lp_ref, ln_ref):
    gp = gp_ref[...]
    gn = gn_ref[...]
    aw = jnp.dot(gp[:, :32], w_ref[...], preferred_element_type=jnp.float32)
    lp_ref[...] = jnp.sum(aw * gp[:, 32:], axis=1, keepdims=True) + bb_ref[...]
    ln_ref[...] = jnp.sum(aw * gn[:, 32:], axis=1, keepdims=True) + bb_ref[...]


def kernel(x, x_shf, edge_index, sub_node_list, positive_ind, negative_ind,
           batch_labels, enc_W1, enc_b1, enc_W2, enc_b2, dec_W1, dec_b1,
           dec_W2, dec_b2, be_W1, be_b1, be_W2, be_b2, bd_W1, bd_b1, bd_W2,
           bd_b2, bil_W, bil_b):
    f32 = jnp.float32
    src2d = edge_index[0].reshape(_NROWS2D, _CHUNK)
    dst2d = edge_index[1].reshape(_NROWS2D, _CHUNK)
    sub2d = sub_node_list.reshape(_NROWS2D, _CHUNK)
    dstpool2d = jnp.repeat(
        jnp.arange(_N, dtype=jnp.int32), _K).reshape(_NROWS2D, _CHUNK)

    eb1 = enc_b1.reshape(1, 64)
    eb2 = enc_b2.reshape(1, 32)
    db1 = dec_b1.reshape(1, 64)
    db2 = dec_b2.reshape(1, _D)
    beb1 = be_b1.reshape(1, -1)
    beb2 = be_b2.reshape(1, -1)
    bdb1 = bd_b1.reshape(1, -1)
    bdb2 = bd_b2.reshape(1, -1)
    bilb = bil_b.reshape(1, 1)
    mid = be_W1.shape[1]

    y1x, y1s = _tc_call(
        _pre_body,
        (jax.ShapeDtypeStruct((_N, 64), f32), jax.ShapeDtypeStruct((_N, 64), f32)),
        [_row_spec(128), _row_spec(128), _full_spec((128, 64))],
        [_row_spec(64), _row_spec(64)],
    )(x, x_shf, enc_W1)
    p0x, p1x = _seg64(y1x, src2d, dst2d)
    p0s, p1s = _seg64(y1s, src2d, dst2d)

    y2 = _tc_call(
        _mid1_body, jax.ShapeDtypeStruct((_N, 64), f32),
        [_row_spec(64), _row_spec(64), _row_spec(64), _row_spec(64),
         _row_spec(64), _row_spec(64), _full_spec((1, 64)),
         _full_spec((64, 32))],
        _row_spec(64),
    )(y1x, y1s, p0x, p1x, p0s, p1s, eb1, enc_W2)

    q0, q1 = _seg64(y2, src2d, dst2d)
    z_both, znb = _tc_call(
        _mid2_body,
        (jax.ShapeDtypeStruct((_N, 64), f32), jax.ShapeDtypeStruct((_N, 48), f32)),
        [_row_spec(64), _row_spec(64), _row_spec(64), _full_spec((1, 32)),
         _row_spec(8), _full_spec((8, mid)), _full_spec((1, mid)),
         _full_spec((mid, 16)), _full_spec((1, 16))],
        [_row_spec(64), _row_spec(48)],
    )(y2, q0, q1, eb2, batch_labels, be_W1, beb1, be_W2, beb2)

    a0, a1 = _seg64(z_both, sub2d, dstpool2d)
    d0, d1 = _seg48(znb, src2d, dst2d)

    zsub_both, logits_batch = _tc_call(
        _poolfin_body,
        (jax.ShapeDtypeStruct((_N, 64), f32), jax.ShapeDtypeStruct((_N, 8), f32)),
        [_row_spec(64), _row_spec(64), _full_spec((32, 16)), _full_spec((1, 16)),
         _full_spec((16, 8)), _full_spec((1, 8))],
        [_row_spec(64), _row_spec(8)],
    )(a0, a1, bd_W1, bdb1, bd_W2, bdb2)

    h_dec = _tc_call(
        _mid3_body, jax.ShapeDtypeStruct((_N, 64), f32),
        [_row_spec(48), _row_spec(48), _row_spec(48),
         _full_spec((48, 64)), _full_spec((1, 64))],
        _row_spec(64),
    )(znb, d0, d1, dec_W1, db1)

    e0, e1 = _seg64(h_dec, src2d, dst2d)
    gp, gn = _pair_gather(zsub_both, positive_ind, negative_ind)

    recon = _tc_call(
        _fin_body, jax.ShapeDtypeStruct((_N, _D), f32),
        [_row_spec(64), _row_spec(64), _row_spec(64),
         _full_spec((64, _D)), _full_spec((1, _D))],
        _row_spec(_D),
    )(h_dec, e0, e1, dec_W2, db2)

    lp, ln = pl.pallas_call(
        _bil_body,
        grid=(1,),
        out_shape=(jax.ShapeDtypeStruct((_P, 1), f32),
                   jax.ShapeDtypeStruct((_P, 1), f32)),
        in_specs=[_full_spec((_P, 64)), _full_spec((_P, 64)),
                  _full_spec((32, 32)), _full_spec((1, 1))],
        out_specs=[_full_spec((_P, 1)), _full_spec((_P, 1))],
    )(gp, gn, bil_W, bilb)

    return (zsub_both[:, :32], zsub_both[:, 32:], recon,
            lp.reshape(-1), ln.reshape(-1), logits_batch)
